# baseline (speedup 1.0000x reference)
import jax
import jax.numpy as jnp
from jax.experimental import pallas as pl
from jax.experimental.pallas import tpu as pltpu

_B_IN = 16
_B_OUT = 8


def _body(x_ref, row_ref, col_ref, out_ref, pos_ref):
    h = x_ref.shape[1]
    w = x_ref.shape[2]
    d = col_ref.shape[1]

    @pl.when((pl.program_id(0) == 0) & (pl.program_id(1) == 0))
    def _build_pos():
        col = col_ref[0:w, :]
        row = row_ref[0:h, :]
        pos_col = jnp.broadcast_to(col[None, :, :], (h, w, d))
        pos_row = jnp.broadcast_to(row[:, None, :], (h, w, d))
        pos_ref[...] = jnp.concatenate([pos_col, pos_row], axis=-1)

    t = pl.program_id(1)
    out_ref[...] = x_ref[pl.ds(t * _B_OUT, _B_OUT)] + pos_ref[...][None]


def kernel(x, row_embed, col_embed):
    b, c2, h, w = x.shape
    xt = jnp.transpose(x, (0, 2, 3, 1))
    grid = (b // _B_IN, _B_IN // _B_OUT)
    out = pl.pallas_call(
        _body,
        grid=grid,
        in_specs=[
            pl.BlockSpec((_B_IN, h, w, c2), lambda g, t: (g, 0, 0, 0)),
            pl.BlockSpec(row_embed.shape, lambda g, t: (0, 0)),
            pl.BlockSpec(col_embed.shape, lambda g, t: (0, 0)),
        ],
        out_specs=pl.BlockSpec(
            (_B_OUT, h, w, c2),
            lambda g, t: (g * (_B_IN // _B_OUT) + t, 0, 0, 0),
        ),
        out_shape=jax.ShapeDtypeStruct((b, h, w, c2), x.dtype),
        scratch_shapes=[pltpu.VMEM((h, w, c2), x.dtype)],
    )(xt, row_embed, col_embed)
    return jnp.transpose(out, (0, 3, 1, 2))


# speedup vs baseline: 1.1973x; 1.1973x over previous
import jax
import jax.numpy as jnp
from jax.experimental import pallas as pl
from jax.experimental.pallas import tpu as pltpu

_B_BLK = 8


def _body(x_ref, row_ref, col_ref, out_ref, pos_ref):
    h = x_ref.shape[1]
    w = x_ref.shape[2]
    d = col_ref.shape[1]

    @pl.when(pl.program_id(0) == 0)
    def _build_pos():
        col = col_ref[0:w, :]
        row = row_ref[0:h, :]
        pos_col = jnp.broadcast_to(col[None, :, :], (h, w, d))
        pos_row = jnp.broadcast_to(row[:, None, :], (h, w, d))
        pos_ref[...] = jnp.concatenate([pos_col, pos_row], axis=-1)

    out_ref[...] = x_ref[...] + pos_ref[...][None]


def kernel(x, row_embed, col_embed):
    b, c2, h, w = x.shape
    xt = jnp.transpose(x, (0, 2, 3, 1))
    grid = (b // _B_BLK,)
    out = pl.pallas_call(
        _body,
        grid=grid,
        in_specs=[
            pl.BlockSpec((_B_BLK, h, w, c2), lambda g: (g, 0, 0, 0)),
            pl.BlockSpec(row_embed.shape, lambda g: (0, 0)),
            pl.BlockSpec(col_embed.shape, lambda g: (0, 0)),
        ],
        out_specs=pl.BlockSpec((_B_BLK, h, w, c2), lambda g: (g, 0, 0, 0)),
        out_shape=jax.ShapeDtypeStruct((b, h, w, c2), x.dtype),
        scratch_shapes=[pltpu.VMEM((h, w, c2), x.dtype)],
    )(xt, row_embed, col_embed)
    return jnp.transpose(out, (0, 3, 1, 2))
